# baseline (speedup 1.0000x reference)
import functools

import jax
import jax.numpy as jnp
from jax import lax
from jax.experimental import pallas as pl
from jax.experimental.pallas import tpu as pltpu
from jax.experimental.pallas import tpu_sc as plsc

_L = 128
_K = 5
_NB = 4


def _build(nrows, D, nc, ns):
    nw = nc * ns // 2
    rows_per_w = nrows // nw
    chunks = rows_per_w // _K
    groups = chunks // _NB
    mesh = plsc.VectorSubcoreMesh(core_axis_name="c", subcore_axis_name="s")

    @functools.partial(
        pl.kernel,
        out_type=jax.ShapeDtypeStruct((nrows, _L, D), jnp.float32),
        mesh=mesh,
        scratch_types=[
            pltpu.VMEM((_NB, _K, _L), jnp.int32),
            pltpu.VMEM((_NB, _K, _L, D), jnp.float32),
            pltpu.SemaphoreType.DMA((_NB,)),
            pltpu.SemaphoreType.DMA((_NB,)),
            pltpu.SemaphoreType.DMA((_NB,)),
        ],
        compiler_params=pltpu.CompilerParams(use_tc_tiling_on_sc=False),
    )
    def emb(idx_hbm, tab_hbm, out_hbm, idx_v, rows_v, sem_idx, sem_g, sem_st):
        sid = lax.axis_index("s")
        wid = sid * nc + lax.axis_index("c")
        base = wid * rows_per_w

        def idx_copy(i, b):
            return pltpu.make_async_copy(
                idx_hbm.at[pl.ds(base + i * _K, _K)], idx_v.at[b], sem_idx.at[b]
            )

        def st_copy(i, b):
            return pltpu.make_async_copy(
                rows_v.at[b], out_hbm.at[pl.ds(base + i * _K, _K)], sem_st.at[b]
            )

        def fire_gathers(b):
            for j in range(_K):
                pltpu.async_copy(
                    tab_hbm.at[idx_v.at[b].at[j]], rows_v.at[b].at[j], sem_g.at[b]
                )

        def wait_gathers(i, b):
            pltpu.make_async_copy(
                out_hbm.at[pl.ds(base + i * _K, _K)], rows_v.at[b], sem_g.at[b]
            ).wait()

        @pl.when(sid < ns // 2)
        def _run():
            for b in range(_NB):
                idx_copy(b, b).start()
            idx_copy(0, 0).wait()
            fire_gathers(0)

            def group(g, carry):
                for b in range(_NB):
                    i = g * _NB + b
                    b1 = (b + 1) % _NB
                    if b < _NB - 1:
                        idx_copy(i + 1, b1).wait()
                        @pl.when(g > 0)
                        def _():
                            st_copy(i + 1 - _NB, b1).wait()
                        fire_gathers(b1)
                    else:
                        @pl.when(g < groups - 1)
                        def _():
                            idx_copy(i + 1, b1).wait()
                            st_copy(i + 1 - _NB, b1).wait()
                            fire_gathers(b1)
                    wait_gathers(i, b)
                    @pl.when(g < groups - 1)
                    def _():
                        idx_copy(i + _NB, b).start()
                    st_copy(i, b).start()
                return carry

            lax.fori_loop(0, groups, group, 0)

            for b in range(_NB):
                st_copy((groups - 1) * _NB + b, b).wait()

    return emb


def kernel(x, weight):
    B, S = x.shape
    _, D = weight.shape
    total = B * S
    nrows = total // _L
    info = plsc.get_sparse_core_info()
    emb = _build(nrows, D, info.num_cores, info.num_subcores)
    idx = x.reshape(nrows, _L).astype(jnp.int32)
    out = emb(idx, weight)
    return out.reshape(B, S, D)


# speedup vs baseline: 1.0139x; 1.0139x over previous
import functools

import jax
import jax.numpy as jnp
from jax import lax
from jax.experimental import pallas as pl
from jax.experimental.pallas import tpu as pltpu
from jax.experimental.pallas import tpu_sc as plsc

_L = 128
_K = 10
_NB = 2


def _build(nrows, D, nc, ns):
    nw = nc * ns
    rows_per_w = nrows // nw
    chunks = rows_per_w // _K
    groups = chunks // _NB
    mesh = plsc.VectorSubcoreMesh(core_axis_name="c", subcore_axis_name="s")

    @functools.partial(
        pl.kernel,
        out_type=jax.ShapeDtypeStruct((nrows, _L, D), jnp.float32),
        mesh=mesh,
        scratch_types=[
            pltpu.VMEM((_NB, _K, _L), jnp.int32),
            pltpu.VMEM((_NB, _K, _L, D), jnp.float32),
            pltpu.SemaphoreType.DMA((_NB,)),
            pltpu.SemaphoreType.DMA((_NB,)),
            pltpu.SemaphoreType.DMA((_NB,)),
        ],
        compiler_params=pltpu.CompilerParams(use_tc_tiling_on_sc=False),
    )
    def emb(idx_hbm, tab_hbm, out_hbm, idx_v, rows_v, sem_idx, sem_g, sem_st):
        wid = lax.axis_index("s") * nc + lax.axis_index("c")
        base = wid * rows_per_w

        def idx_copy(i, b):
            return pltpu.make_async_copy(
                idx_hbm.at[pl.ds(base + i * _K, _K)], idx_v.at[b], sem_idx.at[b]
            )

        def st_copy(i, b):
            return pltpu.make_async_copy(
                rows_v.at[b], out_hbm.at[pl.ds(base + i * _K, _K)], sem_st.at[b]
            )

        def fire_gathers(b):
            for j in range(_K):
                pltpu.async_copy(
                    tab_hbm.at[idx_v.at[b].at[j]], rows_v.at[b].at[j], sem_g.at[b]
                )

        def wait_gathers(i, b):
            pltpu.make_async_copy(
                out_hbm.at[pl.ds(base + i * _K, _K)], rows_v.at[b], sem_g.at[b]
            ).wait()

        for b in range(_NB):
            idx_copy(b, b).start()
        idx_copy(0, 0).wait()
        fire_gathers(0)

        def group(g, carry):
            for b in range(_NB):
                i = g * _NB + b
                b1 = (b + 1) % _NB
                if b < _NB - 1:
                    idx_copy(i + 1, b1).wait()
                    @pl.when(g > 0)
                    def _():
                        st_copy(i + 1 - _NB, b1).wait()
                    fire_gathers(b1)
                else:
                    @pl.when(g < groups - 1)
                    def _():
                        idx_copy(i + 1, b1).wait()
                        st_copy(i + 1 - _NB, b1).wait()
                        fire_gathers(b1)
                wait_gathers(i, b)
                @pl.when(g < groups - 1)
                def _():
                    idx_copy(i + _NB, b).start()
                st_copy(i, b).start()
            return carry

        lax.fori_loop(0, groups, group, 0)

        for b in range(_NB):
            st_copy((groups - 1) * _NB + b, b).wait()

    return emb


def kernel(x, weight):
    B, S = x.shape
    _, D = weight.shape
    total = B * S
    nrows = total // _L
    info = plsc.get_sparse_core_info()
    emb = _build(nrows, D, info.num_cores, info.num_subcores)
    idx = x.reshape(nrows, _L).astype(jnp.int32)
    out = emb(idx, weight)
    return out.reshape(B, S, D)


# speedup vs baseline: 1.0145x; 1.0007x over previous
import functools

import jax
import jax.numpy as jnp
from jax import lax
from jax.experimental import pallas as pl
from jax.experimental.pallas import tpu as pltpu
from jax.experimental.pallas import tpu_sc as plsc

_L = 128
_K = 5
_NB = 4


def _build(nrows, D, nc, ns):
    nw = nc * ns
    rows_per_w = nrows // nw
    chunks = rows_per_w // _K
    groups = chunks // _NB
    mesh = plsc.VectorSubcoreMesh(core_axis_name="c", subcore_axis_name="s")

    @functools.partial(
        pl.kernel,
        out_type=jax.ShapeDtypeStruct((nrows, _L, D), jnp.float32),
        mesh=mesh,
        scratch_types=[
            pltpu.VMEM((_NB, _K, _L), jnp.int32),
            pltpu.VMEM((_NB, _K, _L, D), jnp.float32),
            pltpu.SemaphoreType.DMA((_NB,)),
            pltpu.SemaphoreType.DMA((_NB,)),
            pltpu.SemaphoreType.DMA((_NB,)),
        ],
        compiler_params=pltpu.CompilerParams(use_tc_tiling_on_sc=False),
    )
    def emb(idx_hbm, tab_hbm, out_hbm, idx_v, rows_v, sem_idx, sem_g, sem_st):
        wid = lax.axis_index("s") * nc + lax.axis_index("c")
        base = wid * rows_per_w

        def idx_copy(i, b):
            return pltpu.make_async_copy(
                idx_hbm.at[pl.ds(base + i * _K, _K)], idx_v.at[b], sem_idx.at[b]
            )

        def st_copy(i, b):
            return pltpu.make_async_copy(
                rows_v.at[b], out_hbm.at[pl.ds(base + i * _K, _K)], sem_st.at[b]
            )

        def fire_gathers(b):
            for j in range(_K):
                pltpu.async_copy(
                    tab_hbm.at[idx_v.at[b].at[j]], rows_v.at[b].at[j], sem_g.at[b]
                )

        def wait_gathers(i, b):
            pltpu.make_async_copy(
                out_hbm.at[pl.ds(base + i * _K, _K)], rows_v.at[b], sem_g.at[b]
            ).wait()

        for b in range(_NB):
            idx_copy(b, b).start()
        idx_copy(0, 0).wait()
        fire_gathers(0)

        def group(g, carry):
            for b in range(_NB):
                i = g * _NB + b
                b1 = (b + 1) % _NB
                bp = (b - 1) % _NB
                if b < _NB - 1:
                    idx_copy(i + 1, b1).wait()
                    @pl.when(g > 0)
                    def _():
                        st_copy(i + 1 - _NB, b1).wait()
                    fire_gathers(b1)
                else:
                    @pl.when(g < groups - 1)
                    def _():
                        idx_copy(i + 1, b1).wait()
                        st_copy(i + 1 - _NB, b1).wait()
                        fire_gathers(b1)
                wait_gathers(i, b)
                @pl.when(g < groups - 1)
                def _():
                    idx_copy(i + _NB, b).start()
                if b > 0:
                    st_copy(i - 1, bp).start()
                else:
                    @pl.when(g > 0)
                    def _():
                        st_copy(i - 1, bp).start()
            return carry

        lax.fori_loop(0, groups, group, 0)

        st_copy(chunks - 1, (chunks - 1) % _NB).start()
        for b in range(_NB):
            st_copy(chunks - _NB + b, (chunks - _NB + b) % _NB).wait()

    return emb


def kernel(x, weight):
    B, S = x.shape
    _, D = weight.shape
    total = B * S
    nrows = total // _L
    info = plsc.get_sparse_core_info()
    emb = _build(nrows, D, info.num_cores, info.num_subcores)
    idx = x.reshape(nrows, _L).astype(jnp.int32)
    out = emb(idx, weight)
    return out.reshape(B, S, D)
